# baseline (speedup 1.0000x reference)
import functools

import jax
import jax.numpy as jnp
from jax import lax
from jax.experimental import pallas as pl
from jax.experimental.pallas import tpu as pltpu
from jax.experimental.pallas import tpu_sc as plsc

B = 16384
D = 64
DW = D // 2
EPS = 1e-3

NC = 2
NS = 16
NW = NC * NS
BPW = B // NW
CHUNK = 128
NCHUNK = BPW // CHUNK


def _sc_gather(user_ids, joke_ids, utab_w, jtab_w):
    mesh = plsc.VectorSubcoreMesh(core_axis_name="c", subcore_axis_name="s")

    @functools.partial(
        pl.kernel,
        out_type=jax.ShapeDtypeStruct((B, 2 * DW), jnp.int32),
        mesh=mesh,
        scratch_types=[
            pltpu.VMEM((NCHUNK, CHUNK), jnp.int32),
            pltpu.VMEM((NCHUNK, CHUNK), jnp.int32),
            pltpu.VMEM((BPW, DW), jnp.int32),
            pltpu.VMEM((BPW, DW), jnp.int32),
            pltpu.SemaphoreType.DMA,
        ],
        compiler_params=pltpu.CompilerParams(use_tc_tiling_on_sc=False),
    )
    def gather_kernel(uid_hbm, jid_hbm, utab_hbm, jtab_hbm, out_hbm,
                      uidx_v, jidx_v, urows_v, jrows_v, sem):
        wid = lax.axis_index("s") * NC + lax.axis_index("c")
        base = wid * BPW
        for j in range(NCHUNK):
            pltpu.sync_copy(uid_hbm.at[pl.ds(base + j * CHUNK, CHUNK)],
                            uidx_v.at[j])
            pltpu.sync_copy(jid_hbm.at[pl.ds(base + j * CHUNK, CHUNK)],
                            jidx_v.at[j])
        copies = []
        for j in range(NCHUNK):
            copies.append(pltpu.async_copy(
                utab_hbm.at[uidx_v.at[j]],
                urows_v.at[pl.ds(j * CHUNK, CHUNK)], sem))
            copies.append(pltpu.async_copy(
                jtab_hbm.at[jidx_v.at[j]],
                jrows_v.at[pl.ds(j * CHUNK, CHUNK)], sem))
        for c in copies:
            c.wait()
        pltpu.sync_copy(urows_v, out_hbm.at[pl.ds(base, BPW), pl.ds(0, DW)])
        pltpu.sync_copy(jrows_v, out_hbm.at[pl.ds(base, BPW), pl.ds(DW, DW)])

    return gather_kernel(user_ids, joke_ids, utab_w, jtab_w)


def _mlp_body(x_ref, w1_ref, b1_ref, w2_ref, b2_ref, w3_ref, b3_ref,
              w4_ref, b4_ref, w5_ref, b5_ref, out_ref):
    x = x_ref[...].astype(jnp.float32)
    h = jnp.maximum(jnp.dot(x, w1_ref[...],
                            preferred_element_type=jnp.float32,
                            precision=jax.lax.Precision.HIGHEST) + b1_ref[...], 0.0)
    h = jnp.maximum(jnp.dot(h, w2_ref[...],
                            preferred_element_type=jnp.float32,
                            precision=jax.lax.Precision.HIGHEST) + b2_ref[...], 0.0)
    h = jnp.maximum(jnp.dot(h, w3_ref[...],
                            preferred_element_type=jnp.float32,
                            precision=jax.lax.Precision.HIGHEST) + b3_ref[...], 0.0)
    h = jnp.maximum(jnp.dot(h, w4_ref[...],
                            preferred_element_type=jnp.float32,
                            precision=jax.lax.Precision.HIGHEST) + b4_ref[...], 0.0)
    h = jnp.maximum(jnp.dot(h, w5_ref[...],
                            preferred_element_type=jnp.float32,
                            precision=jax.lax.Precision.HIGHEST) + b5_ref[...], 0.0)
    out_ref[...] = h


def _tc_mlp(x, W1, b1, W2, b2, W3, b3, W4, b4, W5, b5):
    blk = 2048
    grid = (B // blk,)
    full = lambda a: pl.BlockSpec(a.shape, lambda i: (0,) * a.ndim)
    return pl.pallas_call(
        _mlp_body,
        grid=grid,
        in_specs=[
            pl.BlockSpec((blk, 2 * D), lambda i: (i, 0)),
            full(W1), full(b1), full(W2), full(b2), full(W3), full(b3),
            full(W4), full(b4), full(W5), full(b5),
        ],
        out_specs=pl.BlockSpec((blk, 1), lambda i: (i, 0)),
        out_shape=jax.ShapeDtypeStruct((B, 1), jnp.float32),
    )(x, W1, b1, W2, b2, W3, b3, W4, b4, W5, b5)


def kernel(user_ids, joke_ids, user_table, joke_table,
           W1, b1, g1, be1, W2, b2, g2, be2, W3, b3, W4, b4, W5, b5):
    utab_w = jax.lax.bitcast_convert_type(
        user_table.astype(jnp.bfloat16).reshape(-1, DW, 2), jnp.int32)
    jtab_w = jax.lax.bitcast_convert_type(
        joke_table.astype(jnp.bfloat16).reshape(-1, DW, 2), jnp.int32)
    xw = _sc_gather(user_ids.astype(jnp.int32), joke_ids.astype(jnp.int32),
                    utab_w, jtab_w)
    x = jax.lax.bitcast_convert_type(xw, jnp.bfloat16).reshape(B, 2 * D)
    inv = 1.0 / jnp.sqrt(jnp.float32(1.0 + EPS))
    s1 = g1 * inv
    W2f = s1[:, None] * W2
    b2f = be1 @ W2 + b2
    s2 = g2 * inv
    W3f = s2[:, None] * W3
    b3f = be2 @ W3 + b3
    return _tc_mlp(x, W1, b1[None, :], W2f, b2f[None, :], W3f, b3f[None, :],
                   W4, b4[None, :], W5, b5[None, :])


# speedup vs baseline: 2.6329x; 2.6329x over previous
import functools

import jax
import jax.numpy as jnp
from jax import lax
from jax.experimental import pallas as pl
from jax.experimental.pallas import tpu as pltpu
from jax.experimental.pallas import tpu_sc as plsc

B = 16384
D = 64
EPS = 1e-3
NU = 1000000 + 1
NJ = 100000 + 1

NC = 2
NS = 16
NW = NC * NS
BPW = B // NW
CHUNK = 128
NCHUNK = BPW // CHUNK


def _sc_gather(upidx, jpidx, upair, jpair):
    mesh = plsc.VectorSubcoreMesh(core_axis_name="c", subcore_axis_name="s")

    @functools.partial(
        pl.kernel,
        out_type=(
            jax.ShapeDtypeStruct((B, 2 * D), jnp.float32),
            jax.ShapeDtypeStruct((B, 2 * D), jnp.float32),
        ),
        mesh=mesh,
        scratch_types=[
            pltpu.VMEM((NCHUNK, CHUNK), jnp.int32),
            pltpu.VMEM((NCHUNK, CHUNK), jnp.int32),
            pltpu.VMEM((BPW, 2 * D), jnp.float32),
            pltpu.SemaphoreType.DMA,
        ],
    )
    def gather_kernel(uid_hbm, jid_hbm, utab_hbm, jtab_hbm,
                      uout_hbm, jout_hbm, uidx_v, jidx_v, rows_v, sem):
        wid = lax.axis_index("s") * NC + lax.axis_index("c")
        base = wid * BPW
        for j in range(NCHUNK):
            pltpu.sync_copy(uid_hbm.at[pl.ds(base + j * CHUNK, CHUNK)],
                            uidx_v.at[j])
            pltpu.sync_copy(jid_hbm.at[pl.ds(base + j * CHUNK, CHUNK)],
                            jidx_v.at[j])
        copies = []
        for j in range(NCHUNK):
            copies.append(pltpu.async_copy(
                utab_hbm.at[uidx_v.at[j]],
                rows_v.at[pl.ds(j * CHUNK, CHUNK)], sem))
        for c in copies:
            c.wait()
        pltpu.sync_copy(rows_v, uout_hbm.at[pl.ds(base, BPW)])
        copies = []
        for j in range(NCHUNK):
            copies.append(pltpu.async_copy(
                jtab_hbm.at[jidx_v.at[j]],
                rows_v.at[pl.ds(j * CHUNK, CHUNK)], sem))
        for c in copies:
            c.wait()
        pltpu.sync_copy(rows_v, jout_hbm.at[pl.ds(base, BPW)])

    return gather_kernel(upidx, jpidx, upair, jpair)


def _mlp_body(xu_ref, xj_ref, uid_ref, jid_ref, ulast_ref, jlast_ref,
              w1u_ref, w1j_ref, b1_ref, w2_ref, b2_ref, w3_ref, b3_ref,
              w4_ref, b4_ref, w5_ref, b5_ref, out_ref):
    uid = uid_ref[...]
    jid = jid_ref[...]
    up = (uid & 1).astype(jnp.float32)
    jp = (jid & 1).astype(jnp.float32)
    xu = xu_ref[...]
    xj = xj_ref[...]
    u = xu[:, :D] * (1.0 - up) + xu[:, D:] * up
    j = xj[:, :D] * (1.0 - jp) + xj[:, D:] * jp
    u = jnp.where(uid == NU - 1, ulast_ref[...], u)
    j = jnp.where(jid == NJ - 1, jlast_ref[...], j)
    hp = jax.lax.Precision.HIGHEST
    h = jnp.maximum(
        jnp.dot(u, w1u_ref[...], preferred_element_type=jnp.float32,
                precision=hp)
        + jnp.dot(j, w1j_ref[...], preferred_element_type=jnp.float32,
                  precision=hp)
        + b1_ref[...], 0.0)
    h = jnp.maximum(jnp.dot(h, w2_ref[...],
                            preferred_element_type=jnp.float32,
                            precision=hp) + b2_ref[...], 0.0)
    h = jnp.maximum(jnp.dot(h, w3_ref[...],
                            preferred_element_type=jnp.float32,
                            precision=hp) + b3_ref[...], 0.0)
    h = jnp.maximum(jnp.dot(h, w4_ref[...],
                            preferred_element_type=jnp.float32,
                            precision=hp) + b4_ref[...], 0.0)
    h = jnp.maximum(jnp.dot(h, w5_ref[...],
                            preferred_element_type=jnp.float32,
                            precision=hp) + b5_ref[...], 0.0)
    out_ref[...] = h


def _tc_mlp(xu, xj, uid2, jid2, ulast, jlast,
            W1u, W1j, b1, W2, b2, W3, b3, W4, b4, W5, b5):
    blk = 2048
    grid = (B // blk,)
    full = lambda a: pl.BlockSpec(a.shape, lambda i: (0,) * a.ndim)
    return pl.pallas_call(
        _mlp_body,
        grid=grid,
        in_specs=[
            pl.BlockSpec((blk, 2 * D), lambda i: (i, 0)),
            pl.BlockSpec((blk, 2 * D), lambda i: (i, 0)),
            pl.BlockSpec((blk, 1), lambda i: (i, 0)),
            pl.BlockSpec((blk, 1), lambda i: (i, 0)),
            full(ulast), full(jlast),
            full(W1u), full(W1j), full(b1), full(W2), full(b2),
            full(W3), full(b3), full(W4), full(b4), full(W5), full(b5),
        ],
        out_specs=pl.BlockSpec((blk, 1), lambda i: (i, 0)),
        out_shape=jax.ShapeDtypeStruct((B, 1), jnp.float32),
    )(xu, xj, uid2, jid2, ulast, jlast,
      W1u, W1j, b1, W2, b2, W3, b3, W4, b4, W5, b5)


def kernel(user_ids, joke_ids, user_table, joke_table,
           W1, b1, g1, be1, W2, b2, g2, be2, W3, b3, W4, b4, W5, b5):
    upair = user_table[:NU - 1].reshape((NU - 1) // 2, 2 * D)
    jpair = joke_table[:NJ - 1].reshape((NJ - 1) // 2, 2 * D)
    uid = user_ids.astype(jnp.int32)
    jid = joke_ids.astype(jnp.int32)
    upidx = jnp.minimum(uid, NU - 2) // 2
    jpidx = jnp.minimum(jid, NJ - 2) // 2
    xu, xj = _sc_gather(upidx, jpidx, upair, jpair)
    inv = 1.0 / jnp.sqrt(jnp.float32(1.0 + EPS))
    s1 = g1 * inv
    W2f = s1[:, None] * W2
    b2f = be1 @ W2 + b2
    s2 = g2 * inv
    W3f = s2[:, None] * W3
    b3f = be2 @ W3 + b3
    return _tc_mlp(xu, xj, uid[:, None], jid[:, None],
                   user_table[NU - 1][None, :], joke_table[NJ - 1][None, :],
                   W1[:D], W1[D:], b1[None, :], W2f, b2f[None, :],
                   W3f, b3f[None, :], W4, b4[None, :], W5, b5[None, :])
